# initial kernel scaffold (unmeasured)
import jax
import jax.numpy as jnp
from jax import lax
from jax.experimental import pallas as pl
from jax.experimental.pallas import tpu as pltpu

N_DEV = 4
N_TOK = 1024
D_IN = 512
D_OUT = 1024
N_EXP = 16
E_LOC = N_EXP // N_DEV
N_HOPS = N_DEV - 1


def kernel(x, router_W, route_idx, expert_W):
    def body(x_ref, rw_ref, idx_ref, ew_ref, out_ref, comm_ref, send_sems, recv_sems):
        my = lax.axis_index("i")
        left = lax.rem(my + N_DEV - 1, N_DEV)
        right = lax.rem(my + 1, N_DEV)

        barrier = pltpu.get_barrier_semaphore()
        for nbr in (left, right):
            pl.semaphore_signal(
                barrier, inc=1, device_id=(nbr,),
                device_id_type=pl.DeviceIdType.MESH,
            )
        pl.semaphore_wait(barrier, 2)

        xv = x_ref[:, :]
        scores = jnp.dot(xv, rw_ref[:, :], preferred_element_type=jnp.float32)
        p = jnp.exp(scores - jnp.max(scores, axis=-1, keepdims=True))
        p = p / jnp.sum(p, axis=-1, keepdims=True)

        idx = idx_ref[:, :]
        eids = lax.broadcasted_iota(jnp.int32, (N_TOK, N_EXP), 1)
        oh0 = eids == idx[:, 0:1]
        oh1 = eids == idx[:, 1:2]
        g0 = jnp.sum(jnp.where(oh0, p, 0.0), axis=-1, keepdims=True)
        g1 = jnp.sum(jnp.where(oh1, p, 0.0), axis=-1, keepdims=True)
        inv = 1.0 / (g0 + g1)
        gate = jnp.where(oh0, g0 * inv, 0.0) + jnp.where(oh1, g1 * inv, 0.0)
        gate_loc = lax.dynamic_slice(gate, (0, my * E_LOC), (N_TOK, E_LOC))

        acc = jnp.zeros((N_TOK, D_OUT), jnp.float32)
        for k in range(E_LOC):
            acc = acc + jnp.dot(
                gate_loc[:, k:k + 1] * xv, ew_ref[k],
                preferred_element_type=jnp.float32,
            )
        out_ref[:, :] = acc
        comm_ref[0] = acc

        for h in range(N_HOPS):
            rdma = pltpu.make_async_remote_copy(
                src_ref=comm_ref.at[h],
                dst_ref=comm_ref.at[h + 1],
                send_sem=send_sems.at[h],
                recv_sem=recv_sems.at[h],
                device_id=(right,),
                device_id_type=pl.DeviceIdType.MESH,
            )
            rdma.start()
            rdma.wait()
            out_ref[:, :] += comm_ref[h + 1]

    return pl.pallas_call(
        body,
        out_shape=jax.ShapeDtypeStruct((N_TOK, D_OUT), jnp.float32),
        in_specs=[pl.BlockSpec(memory_space=pltpu.VMEM)] * 4,
        out_specs=pl.BlockSpec(memory_space=pltpu.VMEM),
        scratch_shapes=[
            pltpu.VMEM((N_DEV, N_TOK, D_OUT), jnp.float32),
            pltpu.SemaphoreType.DMA((N_HOPS,)),
            pltpu.SemaphoreType.DMA((N_HOPS,)),
        ],
        compiler_params=pltpu.CompilerParams(collective_id=0),
    )(x, router_W, route_idx, expert_W)


# baseline (device time: 163060 ns/iter reference)
import jax
import jax.numpy as jnp
from jax import lax
from jax.experimental import pallas as pl
from jax.experimental.pallas import tpu as pltpu

N_DEV = 4
N_TOK = 1024
D_IN = 512
D_OUT = 1024
N_EXP = 16
E_LOC = N_EXP // N_DEV
N_HOPS = N_DEV - 1


def kernel(x, router_W, route_idx, expert_W):
    def body(x_ref, rw_ref, idx_ref, ew_ref, out_ref, comm_ref, send_sems, recv_sems):
        my = lax.axis_index("i")
        left = lax.rem(my + N_DEV - 1, N_DEV)
        right = lax.rem(my + 1, N_DEV)

        barrier = pltpu.get_barrier_semaphore()
        for nbr in (left, right):
            pl.semaphore_signal(
                barrier, inc=1, device_id=(nbr,),
                device_id_type=pl.DeviceIdType.MESH,
            )
        pl.semaphore_wait(barrier, 2)

        xv = x_ref[:, :]
        scores = jnp.dot(xv, rw_ref[:, :], preferred_element_type=jnp.float32)
        p = jnp.exp(scores - jnp.max(scores, axis=-1, keepdims=True))
        p = p / jnp.sum(p, axis=-1, keepdims=True)

        idx = idx_ref[:, :]
        eids = lax.broadcasted_iota(jnp.int32, (N_TOK, N_EXP), 1)
        oh0 = eids == idx[:, 0:1]
        oh1 = eids == idx[:, 1:2]
        g0 = jnp.sum(jnp.where(oh0, p, 0.0), axis=-1, keepdims=True)
        g1 = jnp.sum(jnp.where(oh1, p, 0.0), axis=-1, keepdims=True)
        inv = 1.0 / (g0 + g1)

        acc = jnp.zeros((N_TOK, D_OUT), jnp.float32)
        for k in range(E_LOC):
            e_k = my * E_LOC + k
            w_k = (
                jnp.where(idx[:, 0:1] == e_k, g0 * inv, 0.0)
                + jnp.where(idx[:, 1:2] == e_k, g1 * inv, 0.0)
            )
            acc = acc + jnp.dot(
                w_k * xv, ew_ref[k],
                preferred_element_type=jnp.float32,
            )
        out_ref[:, :] = acc
        comm_ref[0] = acc

        for h in range(N_HOPS):
            rdma = pltpu.make_async_remote_copy(
                src_ref=comm_ref.at[h],
                dst_ref=comm_ref.at[h + 1],
                send_sem=send_sems.at[h],
                recv_sem=recv_sems.at[h],
                device_id=(right,),
                device_id_type=pl.DeviceIdType.MESH,
            )
            rdma.start()
            rdma.wait()
            out_ref[:, :] += comm_ref[h + 1]

    return pl.pallas_call(
        body,
        out_shape=jax.ShapeDtypeStruct((N_TOK, D_OUT), jnp.float32),
        in_specs=[pl.BlockSpec(memory_space=pltpu.VMEM)] * 4,
        out_specs=pl.BlockSpec(memory_space=pltpu.VMEM),
        scratch_shapes=[
            pltpu.VMEM((N_DEV, N_TOK, D_OUT), jnp.float32),
            pltpu.SemaphoreType.DMA((N_HOPS,)),
            pltpu.SemaphoreType.DMA((N_HOPS,)),
        ],
        compiler_params=pltpu.CompilerParams(collective_id=0),
    )(x, router_W, route_idx, expert_W)


# device time: 93962 ns/iter; 1.7354x vs baseline; 1.7354x over previous
import jax
import jax.numpy as jnp
from jax import lax
from jax.experimental import pallas as pl
from jax.experimental.pallas import tpu as pltpu

N_DEV = 4
N_TOK = 1024
D_IN = 512
D_OUT = 1024
N_EXP = 16
E_LOC = N_EXP // N_DEV
N_HOPS = N_DEV - 1
CHUNK = N_TOK // N_DEV


def kernel(x, router_W, route_idx, expert_W):
    def body(x_ref, rw_ref, idx_ref, ew_ref, out_ref,
             y_ref, rs_send, rs_recv, ag_buf,
             rs_ssem, rs_rsem, ag_ssem, ag_rsem):
        my = lax.axis_index("i")
        left = lax.rem(my + N_DEV - 1, N_DEV)
        right = lax.rem(my + 1, N_DEV)

        barrier = pltpu.get_barrier_semaphore()
        for nbr in (left, right):
            pl.semaphore_signal(
                barrier, inc=1, device_id=(nbr,),
                device_id_type=pl.DeviceIdType.MESH,
            )
        pl.semaphore_wait(barrier, 2)

        xv = x_ref[:, :]
        scores = jnp.dot(xv, rw_ref[:, :], preferred_element_type=jnp.float32)
        p = jnp.exp(scores - jnp.max(scores, axis=-1, keepdims=True))
        p = p / jnp.sum(p, axis=-1, keepdims=True)
        idx = idx_ref[:, :]
        eids = lax.broadcasted_iota(jnp.int32, (N_TOK, N_EXP), 1)
        g0 = jnp.sum(jnp.where(eids == idx[:, 0:1], p, 0.0), axis=-1, keepdims=True)
        g1 = jnp.sum(jnp.where(eids == idx[:, 1:2], p, 0.0), axis=-1, keepdims=True)
        inv = 1.0 / (g0 + g1)

        for k in range(E_LOC):
            e_k = my * E_LOC + k
            w_k = (
                jnp.where(idx[:, 0:1] == e_k, g0 * inv, 0.0)
                + jnp.where(idx[:, 1:2] == e_k, g1 * inv, 0.0)
            )
            y_ref[:, k * D_IN:(k + 1) * D_IN] = w_k * xv
        ew = jnp.reshape(ew_ref[:, :, :], (E_LOC * D_IN, D_OUT))

        def partial_chunk(c):
            yc = y_ref[pl.ds(c * CHUNK, CHUNK), :]
            return jnp.dot(yc, ew, preferred_element_type=jnp.float32)

        def hop(src, dst, ssem, rsem):
            return pltpu.make_async_remote_copy(
                src_ref=src, dst_ref=dst, send_sem=ssem, recv_sem=rsem,
                device_id=(right,), device_id_type=pl.DeviceIdType.MESH,
            )

        rs = []
        rs_send[0, :, :] = partial_chunk(my)
        r = hop(rs_send.at[0], rs_recv.at[0], rs_ssem.at[0], rs_rsem.at[0])
        r.start()
        rs.append(r)
        ag = []
        for s in range(1, N_DEV):
            acc = partial_chunk(lax.rem(my + (N_DEV - s), N_DEV))
            rs[s - 1].wait_recv()
            merged = rs_recv[s - 1] + acc
            if s < N_DEV - 1:
                rs_send[s, :, :] = merged
                r = hop(rs_send.at[s], rs_recv.at[s],
                        rs_ssem.at[s], rs_rsem.at[s])
                r.start()
                rs.append(r)
            else:
                ag_buf[0, :, :] = merged
                r = hop(ag_buf.at[0], ag_buf.at[1], ag_ssem.at[0], ag_rsem.at[0])
                r.start()
                ag.append(r)
                out_ref[pl.ds(lax.rem(my + 1, N_DEV) * CHUNK, CHUNK), :] = merged

        for h in range(N_HOPS):
            ag[h].wait_recv()
            if h < N_HOPS - 1:
                r = hop(ag_buf.at[h + 1], ag_buf.at[h + 2],
                        ag_ssem.at[h + 1], ag_rsem.at[h + 1])
                r.start()
                ag.append(r)
            c = lax.rem(my + (N_DEV - h), N_DEV)
            out_ref[pl.ds(c * CHUNK, CHUNK), :] = ag_buf[h + 1]

        for r in rs + ag:
            r.wait_send()

    return pl.pallas_call(
        body,
        out_shape=jax.ShapeDtypeStruct((N_TOK, D_OUT), jnp.float32),
        in_specs=[pl.BlockSpec(memory_space=pltpu.VMEM)] * 4,
        out_specs=pl.BlockSpec(memory_space=pltpu.VMEM),
        scratch_shapes=[
            pltpu.VMEM((N_TOK, E_LOC * D_IN), jnp.float32),
            pltpu.VMEM((N_HOPS, CHUNK, D_OUT), jnp.float32),
            pltpu.VMEM((N_HOPS, CHUNK, D_OUT), jnp.float32),
            pltpu.VMEM((N_DEV, CHUNK, D_OUT), jnp.float32),
            pltpu.SemaphoreType.DMA((N_HOPS,)),
            pltpu.SemaphoreType.DMA((N_HOPS,)),
            pltpu.SemaphoreType.DMA((N_HOPS,)),
            pltpu.SemaphoreType.DMA((N_HOPS,)),
        ],
        compiler_params=pltpu.CompilerParams(collective_id=0),
    )(x, router_W, route_idx, expert_W)


# device time: 60136 ns/iter; 2.7115x vs baseline; 1.5625x over previous
import jax
import jax.numpy as jnp
from jax import lax
from jax.experimental import pallas as pl
from jax.experimental.pallas import tpu as pltpu

N_DEV = 4
N_TOK = 1024
D_IN = 512
D_OUT = 1024
N_EXP = 16
E_LOC = N_EXP // N_DEV
N_HOPS = N_DEV - 1
CHUNK = N_TOK // N_DEV
COLS = D_OUT // 2


def kernel(x, router_W, route_idx, expert_W):
    def body(x_ref, rw_ref, idx_ref, ew_ref, out_ref,
             y_ref, rs_send, rs_recv, ag_buf,
             rs_ssem, rs_rsem, ag_ssem, ag_rsem):
        my = lax.axis_index("i")
        left = lax.rem(my + N_DEV - 1, N_DEV)
        right = lax.rem(my + 1, N_DEV)
        peer = (right, left)

        barrier = pltpu.get_barrier_semaphore()
        for nbr in (left, right):
            pl.semaphore_signal(
                barrier, inc=1, device_id=(nbr,),
                device_id_type=pl.DeviceIdType.MESH,
            )
        pl.semaphore_wait(barrier, 2)

        xv = x_ref[:, :]
        scores = jnp.dot(xv, rw_ref[:, :], preferred_element_type=jnp.float32)
        p = jnp.exp(scores - jnp.max(scores, axis=-1, keepdims=True))
        p = p / jnp.sum(p, axis=-1, keepdims=True)
        idx = idx_ref[:, :]
        eids = lax.broadcasted_iota(jnp.int32, (N_TOK, N_EXP), 1)
        g0 = jnp.sum(jnp.where(eids == idx[:, 0:1], p, 0.0), axis=-1, keepdims=True)
        g1 = jnp.sum(jnp.where(eids == idx[:, 1:2], p, 0.0), axis=-1, keepdims=True)
        inv = 1.0 / (g0 + g1)

        for k in range(E_LOC):
            e_k = my * E_LOC + k
            w_k = (
                jnp.where(idx[:, 0:1] == e_k, g0 * inv, 0.0)
                + jnp.where(idx[:, 1:2] == e_k, g1 * inv, 0.0)
            )
            y_ref[:, k * D_IN:(k + 1) * D_IN] = w_k * xv
        ew = jnp.reshape(ew_ref[:, :, :], (E_LOC * D_IN, D_OUT))

        def pchunk(c, ew_cols):
            yc = y_ref[pl.ds(c * CHUNK, CHUNK), :]
            return jnp.dot(yc, ew_cols, preferred_element_type=jnp.float32)

        def cidx(r, s):
            return lax.rem(my + (N_DEV - s if r == 0 else s), N_DEV)

        def rs_hop(r, s):
            return pltpu.make_async_remote_copy(
                src_ref=rs_send.at[r, s], dst_ref=rs_recv.at[r, s],
                send_sem=rs_ssem.at[r, s], recv_sem=rs_rsem.at[r, s],
                device_id=(peer[r],), device_id_type=pl.DeviceIdType.MESH,
            )

        def ag_hop(r, h):
            return pltpu.make_async_remote_copy(
                src_ref=ag_buf.at[r, h], dst_ref=ag_buf.at[r, h + 1],
                send_sem=ag_ssem.at[r, h], recv_sem=ag_rsem.at[r, h],
                device_id=(peer[r],), device_id_type=pl.DeviceIdType.MESH,
            )

        p0 = pchunk(my, ew)
        rs = {0: [], 1: []}
        ag = {0: [], 1: []}
        for r in (0, 1):
            rs_send[r, 0, :, :] = p0[:, r * COLS:(r + 1) * COLS]
            d = rs_hop(r, 0)
            d.start()
            rs[r].append(d)
        for s in range(1, N_DEV):
            acc = {r: pchunk(cidx(r, s), ew[:, r * COLS:(r + 1) * COLS])
                   for r in (0, 1)}
            for r in (0, 1):
                rs[r][s - 1].wait_recv()
                merged = rs_recv[r, s - 1] + acc[r]
                if s < N_DEV - 1:
                    rs_send[r, s, :, :] = merged
                    d = rs_hop(r, s)
                    d.start()
                    rs[r].append(d)
                else:
                    ag_buf[r, 0, :, :] = merged
                    d = ag_hop(r, 0)
                    d.start()
                    ag[r].append(d)
                    out_ref[pl.ds(cidx(r, 3) * CHUNK, CHUNK),
                            r * COLS:(r + 1) * COLS] = merged

        for h in range(N_HOPS):
            for r in (0, 1):
                ag[r][h].wait_recv()
                if h < N_HOPS - 1:
                    d = ag_hop(r, h + 1)
                    d.start()
                    ag[r].append(d)
                out_ref[pl.ds(cidx(r, h) * CHUNK, CHUNK),
                        r * COLS:(r + 1) * COLS] = ag_buf[r, h + 1]

        for r in (0, 1):
            for d in rs[r] + ag[r]:
                d.wait_send()

    return pl.pallas_call(
        body,
        out_shape=jax.ShapeDtypeStruct((N_TOK, D_OUT), jnp.float32),
        in_specs=[pl.BlockSpec(memory_space=pltpu.VMEM)] * 4,
        out_specs=pl.BlockSpec(memory_space=pltpu.VMEM),
        scratch_shapes=[
            pltpu.VMEM((N_TOK, E_LOC * D_IN), jnp.float32),
            pltpu.VMEM((2, N_HOPS, CHUNK, COLS), jnp.float32),
            pltpu.VMEM((2, N_HOPS, CHUNK, COLS), jnp.float32),
            pltpu.VMEM((2, N_DEV, CHUNK, COLS), jnp.float32),
            pltpu.SemaphoreType.DMA((2, N_HOPS)),
            pltpu.SemaphoreType.DMA((2, N_HOPS)),
            pltpu.SemaphoreType.DMA((2, N_HOPS)),
            pltpu.SemaphoreType.DMA((2, N_HOPS)),
        ],
        compiler_params=pltpu.CompilerParams(collective_id=0),
    )(x, router_W, route_idx, expert_W)


# device time: 42663 ns/iter; 3.8220x vs baseline; 1.4096x over previous
import jax
import jax.numpy as jnp
from jax import lax
from jax.experimental import pallas as pl
from jax.experimental.pallas import tpu as pltpu

N_DEV = 4
N_TOK = 1024
D_IN = 512
D_OUT = 1024
N_EXP = 16
E_LOC = N_EXP // N_DEV
N_HOPS = N_DEV - 1
CHUNK = N_TOK // N_DEV
COLS = D_OUT // 2


def kernel(x, router_W, route_idx, expert_W):
    def body(x_ref, rw_ref, idx_ref, ew_ref, out_ref,
             y_ref, rs_send, rs_recv, ag_buf,
             rs_ssem, rs_rsem, ag_ssem, ag_rsem):
        my = lax.axis_index("i")
        left = lax.rem(my + N_DEV - 1, N_DEV)
        right = lax.rem(my + 1, N_DEV)
        peer = (right, left)

        barrier = pltpu.get_barrier_semaphore()
        for nbr in (left, right):
            pl.semaphore_signal(
                barrier, inc=1, device_id=(nbr,),
                device_id_type=pl.DeviceIdType.MESH,
            )
        pl.semaphore_wait(barrier, 2)

        xv = x_ref[:, :]
        scores = jnp.dot(xv, rw_ref[:, :], preferred_element_type=jnp.float32)
        p = jnp.exp(scores - jnp.max(scores, axis=-1, keepdims=True))
        p = p / jnp.sum(p, axis=-1, keepdims=True)
        idx = idx_ref[:, :]
        eids = lax.broadcasted_iota(jnp.int32, (N_TOK, N_EXP), 1)
        g0 = jnp.sum(jnp.where(eids == idx[:, 0:1], p, 0.0), axis=-1, keepdims=True)
        g1 = jnp.sum(jnp.where(eids == idx[:, 1:2], p, 0.0), axis=-1, keepdims=True)
        inv = 1.0 / (g0 + g1)

        for k in range(E_LOC):
            e_k = my * E_LOC + k
            w_k = (
                jnp.where(idx[:, 0:1] == e_k, g0 * inv, 0.0)
                + jnp.where(idx[:, 1:2] == e_k, g1 * inv, 0.0)
            )
            y_ref[:, k * D_IN:(k + 1) * D_IN] = (w_k * xv).astype(jnp.bfloat16)
        ew = jnp.reshape(ew_ref[:, :, :], (E_LOC * D_IN, D_OUT)).astype(jnp.bfloat16)

        def pchunk(c, ew_cols):
            yc = y_ref[pl.ds(c * CHUNK, CHUNK), :]
            return jnp.dot(yc, ew_cols, preferred_element_type=jnp.float32)

        def cidx(r, s):
            return lax.rem(my + (N_DEV - s if r == 0 else s), N_DEV)

        def rs_hop(r, s):
            return pltpu.make_async_remote_copy(
                src_ref=rs_send.at[r, s], dst_ref=rs_recv.at[r, s],
                send_sem=rs_ssem.at[r, s], recv_sem=rs_rsem.at[r, s],
                device_id=(peer[r],), device_id_type=pl.DeviceIdType.MESH,
            )

        def ag_hop(r, h):
            return pltpu.make_async_remote_copy(
                src_ref=ag_buf.at[r, h], dst_ref=ag_buf.at[r, h + 1],
                send_sem=ag_ssem.at[r, h], recv_sem=ag_rsem.at[r, h],
                device_id=(peer[r],), device_id_type=pl.DeviceIdType.MESH,
            )

        p0 = pchunk(my, ew)
        rs = {0: [], 1: []}
        ag = {0: [], 1: []}
        for r in (0, 1):
            rs_send[r, 0, :, :] = p0[:, r * COLS:(r + 1) * COLS].astype(jnp.bfloat16)
            d = rs_hop(r, 0)
            d.start()
            rs[r].append(d)
        for s in range(1, N_DEV):
            acc = {r: pchunk(cidx(r, s), ew[:, r * COLS:(r + 1) * COLS])
                   for r in (0, 1)}
            for r in (0, 1):
                rs[r][s - 1].wait_recv()
                merged = rs_recv[r, s - 1].astype(jnp.float32) + acc[r]
                if s < N_DEV - 1:
                    rs_send[r, s, :, :] = merged.astype(jnp.bfloat16)
                    d = rs_hop(r, s)
                    d.start()
                    rs[r].append(d)
                else:
                    ag_buf[r, 0, :, :] = merged.astype(jnp.bfloat16)
                    d = ag_hop(r, 0)
                    d.start()
                    ag[r].append(d)
                    out_ref[pl.ds(cidx(r, 3) * CHUNK, CHUNK),
                            r * COLS:(r + 1) * COLS] = merged

        for h in range(N_HOPS):
            for r in (0, 1):
                ag[r][h].wait_recv()
                if h < N_HOPS - 1:
                    d = ag_hop(r, h + 1)
                    d.start()
                    ag[r].append(d)
                out_ref[pl.ds(cidx(r, h) * CHUNK, CHUNK),
                        r * COLS:(r + 1) * COLS] = (
                            ag_buf[r, h + 1].astype(jnp.float32))

        for r in (0, 1):
            for d in rs[r] + ag[r]:
                d.wait_send()

    return pl.pallas_call(
        body,
        out_shape=jax.ShapeDtypeStruct((N_TOK, D_OUT), jnp.float32),
        in_specs=[pl.BlockSpec(memory_space=pltpu.VMEM)] * 4,
        out_specs=pl.BlockSpec(memory_space=pltpu.VMEM),
        scratch_shapes=[
            pltpu.VMEM((N_TOK, E_LOC * D_IN), jnp.bfloat16),
            pltpu.VMEM((2, N_HOPS, CHUNK, COLS), jnp.bfloat16),
            pltpu.VMEM((2, N_HOPS, CHUNK, COLS), jnp.bfloat16),
            pltpu.VMEM((2, N_DEV, CHUNK, COLS), jnp.bfloat16),
            pltpu.SemaphoreType.DMA((2, N_HOPS)),
            pltpu.SemaphoreType.DMA((2, N_HOPS)),
            pltpu.SemaphoreType.DMA((2, N_HOPS)),
            pltpu.SemaphoreType.DMA((2, N_HOPS)),
        ],
        compiler_params=pltpu.CompilerParams(collective_id=0),
    )(x, router_W, route_idx, expert_W)
